# baseline (speedup 1.0000x reference)
import functools

import jax
import jax.numpy as jnp
from jax import lax
from jax.experimental import pallas as pl
from jax.experimental.pallas import tpu as pltpu
from jax.experimental.pallas import tpu_sc as plsc

NC = 2
NS = 16
L = 16
NW = NC * NS

ROWS = 8192
COLS = 2048
ROWS_PER_W = ROWS // NW
CHUNK_ROWS = 8
N_CHUNKS = ROWS_PER_W // CHUNK_ROWS
BLOCK = 64
BLOCKS_PER_ROW = COLS // BLOCK


def _flush(cb):
    m = jnp.int32(0xFF)
    return (cb & m) + ((cb >> 8) & m) + ((cb >> 16) & m) + ((cb >> 24) & m)


def _body(in_h, out_h, m0_h, m1_h, s0_h, s1_h, c0_h, c1_h,
          in_v0, in_v1, out_v0, out_v1, m0_v0, m0_v1, m1_v0, m1_v1,
          s0_v, s1_v, c0_v, c1_v, sem0, sem1):
    wid = lax.axis_index("s") * NC + lax.axis_index("c")

    iota = lax.broadcasted_iota(jnp.int32, (L,), 0)
    cidx = [iota * 4 + p for p in range(4)]

    bufs = [
        (in_v0, out_v0, m0_v0, m1_v0, sem0),
        (in_v1, out_v1, m0_v1, m1_v1, sem1),
    ]

    ob = wid // NS
    lrow = (wid % NS) * ROWS_PER_W

    def _copies(c, buf):
        in_b, out_b, m0_b, m1_b, sem = buf
        row = pl.multiple_of(lrow + c * CHUNK_ROWS, CHUNK_ROWS)
        sl = pl.ds(row, CHUNK_ROWS)
        return [
            pltpu.make_async_copy(in_h.at[ob, sl, :], in_b, sem),
            pltpu.make_async_copy(out_h.at[ob, sl, :], out_b, sem),
            pltpu.make_async_copy(m0_h.at[ob, sl, :], m0_b, sem),
            pltpu.make_async_copy(m1_h.at[ob, sl, :], m1_b, sem),
        ]

    def _fire(c, buf):
        for cp in _copies(c, buf):
            cp.start()

    def _wait(c, buf):
        for cp in _copies(c, buf):
            cp.wait()

    _fire(0, bufs[0])
    _fire(1, bufs[1])

    def _chunk_compute(buf, carry):
        in_b, out_b, m0_b, m1_b, _ = buf

        def row_loop(r, acc):
            def blk(j, acc):
                s0, s1, cb0, cb1 = acc
                base = j * BLOCK
                mw0 = m0_b[r, pl.ds(j * (BLOCK // 4), L)]
                mw1 = m1_b[r, pl.ds(j * (BLOCK // 4), L)]
                cb0 = cb0 + mw0
                cb1 = cb1 + mw1
                inb = in_b.at[r, pl.ds(base, BLOCK)]
                outb = out_b.at[r, pl.ds(base, BLOCK)]
                for p in range(4):
                    a = plsc.load_gather(inb, [cidx[p]])
                    o = plsc.load_gather(outb, [cidx[p]])
                    d = jnp.abs(a - o)
                    bit = jnp.int32(1 << (8 * p))
                    s0 = s0 + jnp.where((mw0 & bit) == 0, d, 0.0)
                    s1 = s1 + jnp.where((mw1 & bit) == 0, d, 0.0)
                return s0, s1, cb0, cb1

            return lax.fori_loop(0, BLOCKS_PER_ROW, blk, acc)

        s0, s1, c0, c1 = carry
        zi = jnp.zeros((L,), jnp.int32)
        for half in range(2):
            acc = (s0, s1, zi, zi)
            for r in range(half * 4, half * 4 + 4):
                acc = row_loop(r, acc)
            s0, s1, cb0, cb1 = acc
            c0 = c0 + _flush(cb0)
            c1 = c1 + _flush(cb1)
        return s0, s1, c0, c1

    def step(i, carry):
        for b in range(2):
            c = 2 * i + b
            _wait(c, bufs[b])
            carry = _chunk_compute(bufs[b], carry)

            @pl.when(c + 2 < N_CHUNKS)
            def _():
                _fire(c + 2, bufs[b])
        return carry

    zf = jnp.zeros((L,), jnp.float32)
    zi = jnp.zeros((L,), jnp.int32)
    s0, s1, c0, c1 = lax.fori_loop(0, N_CHUNKS // 2, step, (zf, zf, zi, zi))

    s0_v[...] = s0
    s1_v[...] = s1
    c0_v[...] = c0
    c1_v[...] = c1
    pltpu.sync_copy(s0_v, s0_h.at[wid])
    pltpu.sync_copy(s1_v, s1_h.at[wid])
    pltpu.sync_copy(c0_v, c0_h.at[wid])
    pltpu.sync_copy(c1_v, c1_h.at[wid])


@functools.partial(
    pl.kernel,
    out_type=[
        jax.ShapeDtypeStruct((NW, L), jnp.float32),
        jax.ShapeDtypeStruct((NW, L), jnp.float32),
        jax.ShapeDtypeStruct((NW, L), jnp.int32),
        jax.ShapeDtypeStruct((NW, L), jnp.int32),
    ],
    mesh=plsc.VectorSubcoreMesh(
        core_axis_name="c", subcore_axis_name="s", num_cores=NC, num_subcores=NS
    ),
    compiler_params=pltpu.CompilerParams(
        needs_layout_passes=False, use_tc_tiling_on_sc=True
    ),
    scratch_types=[
        pltpu.VMEM((CHUNK_ROWS, COLS), jnp.float32),
        pltpu.VMEM((CHUNK_ROWS, COLS), jnp.float32),
        pltpu.VMEM((CHUNK_ROWS, COLS), jnp.float32),
        pltpu.VMEM((CHUNK_ROWS, COLS), jnp.float32),
        pltpu.VMEM((CHUNK_ROWS, COLS // 4), jnp.int32),
        pltpu.VMEM((CHUNK_ROWS, COLS // 4), jnp.int32),
        pltpu.VMEM((CHUNK_ROWS, COLS // 4), jnp.int32),
        pltpu.VMEM((CHUNK_ROWS, COLS // 4), jnp.int32),
        pltpu.VMEM((L,), jnp.float32),
        pltpu.VMEM((L,), jnp.float32),
        pltpu.VMEM((L,), jnp.int32),
        pltpu.VMEM((L,), jnp.int32),
        pltpu.SemaphoreType.DMA,
        pltpu.SemaphoreType.DMA,
    ],
)
def _mask_loss_sc(in_h, out_h, m0_h, m1_h, s0_h, s1_h, c0_h, c1_h, *rest):
    _body(in_h, out_h, m0_h, m1_h, s0_h, s1_h, c0_h, c1_h, *rest)


def kernel(input, output, mask0, mask1):
    n = input.size
    m0 = mask0.view(jnp.int32)
    m1 = mask1.view(jnp.int32)
    s0, s1, c0, c1 = _mask_loss_sc(input, output, m0, m1)
    nf = jnp.float32(n)
    cnt0 = nf - c0.sum().astype(jnp.float32)
    cnt1 = nf - c1.sum().astype(jnp.float32)
    return s0.sum() / cnt0 + s1.sum() / cnt1
